# baseline (speedup 1.0000x reference)
import functools
import math

import jax
import jax.numpy as jnp
from jax import lax
from jax.experimental import pallas as pl
from jax.experimental.pallas import tpu as pltpu

KS = 7
PAD = 3
NPOOL = 21
LN_EPS = 1e-6
_GC = math.sqrt(2.0 / math.pi)


def _gelu(z):
    return 0.5 * z * (1.0 + jnp.tanh(_GC * (z + 0.044715 * z * z * z)))


def _one_block(y, pad_ref, dw_w, dw_b, w1b, b1, w2b, b2):
    H, W, C = y.shape
    pad_ref[PAD:PAD + H, PAD:PAD + W, :] = y

    acc = jnp.zeros((H, W, C), jnp.float32)
    for dx in range(KS):
        win = pad_ref[:, dx:dx + W, :]
        for dy in range(KS):
            acc = acc + win[dy:dy + H] * dw_w[dy, dx]
    conv = acc + dw_b

    mu = jnp.mean(conv, axis=-1, keepdims=True)
    var = jnp.mean(jnp.square(conv - mu), axis=-1, keepdims=True)
    xn = (conv - mu) * lax.rsqrt(var + LN_EPS)

    th = 8 if H % 8 == 0 else H
    outs = []
    for r0 in range(0, H, th):
        hb = xn[r0:r0 + th].reshape(th * W, C).astype(jnp.bfloat16)
        z = jnp.dot(hb, w1b, preferred_element_type=jnp.float32) + b1
        z = _gelu(z).astype(jnp.bfloat16)
        o = jnp.dot(z, w2b, preferred_element_type=jnp.float32) + b2
        outs.append(y[r0:r0 + th] + o.reshape(th, W, C))
    return outs[0] if len(outs) == 1 else jnp.concatenate(outs, axis=0)


def _fused_kernel(x_ref, *refs, num_blocks):
    blk = [refs[6 * b:6 * (b + 1)] for b in range(num_blocks)]
    wc_ref, bc_ref = refs[6 * num_blocks], refs[6 * num_blocks + 1]
    pooled_ref = refs[6 * num_blocks + 2]
    logits_ref = refs[6 * num_blocks + 3]
    pad_ref = refs[6 * num_blocks + 4]

    pad_ref[...] = jnp.zeros(pad_ref.shape, pad_ref.dtype)

    y = x_ref[0]
    for b in range(num_blocks):
        dw_w, dw_b, w1b, b1, w2b, b2 = (r[...] for r in blk[b])
        y = _one_block(y, pad_ref, dw_w, dw_b, w1b, b1, w2b, b2)

    H, W, C = y.shape
    bh, bw = H // 4, W // 4
    p4 = []
    for j in range(4):
        colmax = jnp.max(y[:, j * bw:(j + 1) * bw, :], axis=1)
        for i in range(4):
            p4.append(jnp.max(colmax[i * bh:(i + 1) * bh], axis=0,
                              keepdims=True))
    p4 = [p4[j * 4 + i] for i in range(4) for j in range(4)]
    p2 = [jnp.maximum(jnp.maximum(p4[8 * i + 2 * j], p4[8 * i + 2 * j + 1]),
                      jnp.maximum(p4[8 * i + 2 * j + 4], p4[8 * i + 2 * j + 5]))
          for i in range(2) for j in range(2)]
    p1 = jnp.maximum(jnp.maximum(p2[0], p2[1]), jnp.maximum(p2[2], p2[3]))
    rows = [p1] + p2 + p4

    pooled_ref[0] = jnp.concatenate(rows, axis=0)

    flat = jnp.concatenate(rows, axis=1)
    mu = jnp.mean(flat, axis=-1, keepdims=True)
    var = jnp.mean(jnp.square(flat - mu), axis=-1, keepdims=True)
    fn = ((flat - mu) * lax.rsqrt(var + LN_EPS)).astype(jnp.bfloat16)
    logits_ref[0] = (jnp.dot(fn, wc_ref[...],
                             preferred_element_type=jnp.float32) + bc_ref[...])


def _fold_block(dw_w, dw_b, ln_g, ln_b, w1, b1, w2, b2, scale):
    w1f = (ln_g.reshape(-1, 1) * w1)
    b1f = b1 + ln_b @ w1
    w2f = w2 * scale
    b2f = b2 * scale
    return (dw_w, dw_b, w1f.astype(jnp.bfloat16), b1f,
            w2f.astype(jnp.bfloat16), b2f)


def kernel(x, dw_w_0, dw_b_0, ln_g_0, ln_b_0, w1_0, b1_0, w2_0, b2_0, scale_0,
           dw_w_1, dw_b_1, ln_g_1, ln_b_1, w1_1, b1_1, w2_1, b2_1, scale_1,
           dw_w_2, dw_b_2, ln_g_2, ln_b_2, w1_2, b1_2, w2_2, b2_2, scale_2,
           cls_ln_g, cls_ln_b, cls_w, cls_b):
    B, C, H, W = x.shape
    F = NPOOL * C
    nc = cls_w.shape[1]
    ncp = ((nc + 127) // 128) * 128

    xh = jnp.transpose(x, (0, 2, 3, 1)).astype(jnp.float32)

    flat = []
    flat += _fold_block(dw_w_0, dw_b_0, ln_g_0, ln_b_0, w1_0, b1_0, w2_0, b2_0, scale_0)
    flat += _fold_block(dw_w_1, dw_b_1, ln_g_1, ln_b_1, w1_1, b1_1, w2_1, b2_1, scale_1)
    flat += _fold_block(dw_w_2, dw_b_2, ln_g_2, ln_b_2, w1_2, b1_2, w2_2, b2_2, scale_2)

    w_pm = cls_w.reshape(C, NPOOL, nc).transpose(1, 0, 2).reshape(F, nc)
    g_pm = cls_ln_g.reshape(C, NPOOL).T.reshape(F, 1)
    b_pm = cls_ln_b.reshape(C, NPOOL).T.reshape(1, F)
    wc = jnp.pad(g_pm * w_pm, ((0, 0), (0, ncp - nc))).astype(jnp.bfloat16)
    bc = jnp.pad(cls_b + b_pm @ w_pm, ((0, 0), (0, ncp - nc)))
    flat += [wc, bc]

    hp = H + 2 * PAD
    wp = ((W + 2 * PAD + 7) // 8) * 8

    def whole(a):
        return pl.BlockSpec(a.shape, lambda b, n=a.ndim: (0,) * n)

    pooled, logits = pl.pallas_call(
        functools.partial(_fused_kernel, num_blocks=3),
        out_shape=(jax.ShapeDtypeStruct((B, NPOOL, C), jnp.float32),
                   jax.ShapeDtypeStruct((B, 1, ncp), jnp.float32)),
        grid=(B,),
        in_specs=[pl.BlockSpec((1, H, W, C), lambda b: (b, 0, 0, 0))]
                 + [whole(a) for a in flat],
        out_specs=(pl.BlockSpec((1, NPOOL, C), lambda b: (b, 0, 0)),
                   pl.BlockSpec((1, 1, ncp), lambda b: (b, 0, 0))),
        scratch_shapes=[pltpu.VMEM((hp, wp, C), jnp.float32)],
        compiler_params=pltpu.CompilerParams(
            dimension_semantics=("parallel",),
            vmem_limit_bytes=48 * 1024 * 1024),
    )(xh, *flat)

    return logits[:, 0, :nc], jnp.transpose(pooled, (0, 2, 1))


# speedup vs baseline: 1.0227x; 1.0227x over previous
import functools
import math

import jax
import jax.numpy as jnp
from jax import lax
from jax.experimental import pallas as pl
from jax.experimental.pallas import tpu as pltpu

KS = 7
PAD = 3
NPOOL = 21
LN_EPS = 1e-6
_GC = math.sqrt(2.0 / math.pi)


def _gelu(z):
    return 0.5 * z * (1.0 + jnp.tanh(_GC * (z + 0.044715 * z * z * z)))


def _one_block(y, pad_ref, dw_w, dw_b, w1b, b1, w2b, b2):
    H, W, C = y.shape
    pad_ref[PAD:PAD + H, PAD:PAD + W, :] = y

    acc = jnp.zeros((H, W, C), jnp.float32)
    for dx in range(KS):
        win = pad_ref[:, dx:dx + W, :]
        for dy in range(KS):
            acc = acc + win[dy:dy + H] * dw_w[dy, dx]
    conv = acc + dw_b

    mu = jnp.mean(conv, axis=-1, keepdims=True)
    var = jnp.mean(jnp.square(conv - mu), axis=-1, keepdims=True)
    xn = (conv - mu) * lax.rsqrt(var + LN_EPS)

    th = 8 if H % 8 == 0 else H
    outs = []
    for r0 in range(0, H, th):
        hb = xn[r0:r0 + th].reshape(th * W, C).astype(jnp.bfloat16)
        z = jnp.dot(hb, w1b, preferred_element_type=jnp.float32) + b1
        z = _gelu(z).astype(jnp.bfloat16)
        o = jnp.dot(z, w2b, preferred_element_type=jnp.float32) + b2
        outs.append(y[r0:r0 + th] + o.reshape(th, W, C))
    return outs[0] if len(outs) == 1 else jnp.concatenate(outs, axis=0)


def _blocks_spp_kernel(x_ref, *refs, num_blocks):
    blk = [refs[6 * b:6 * (b + 1)] for b in range(num_blocks)]
    pooled_ref = refs[6 * num_blocks]
    pad_ref = refs[6 * num_blocks + 1]

    pad_ref[...] = jnp.zeros(pad_ref.shape, pad_ref.dtype)

    y = x_ref[0]
    for b in range(num_blocks):
        dw_w, dw_b, w1b, b1, w2b, b2 = (r[...] for r in blk[b])
        y = _one_block(y, pad_ref, dw_w, dw_b, w1b, b1, w2b, b2)

    H, W, C = y.shape
    bh, bw = H // 4, W // 4
    p4 = []
    for j in range(4):
        colmax = jnp.max(y[:, j * bw:(j + 1) * bw, :], axis=1)
        for i in range(4):
            p4.append(jnp.max(colmax[i * bh:(i + 1) * bh], axis=0,
                              keepdims=True))
    p4 = [p4[j * 4 + i] for i in range(4) for j in range(4)]
    p2 = [jnp.maximum(jnp.maximum(p4[8 * i + 2 * j], p4[8 * i + 2 * j + 1]),
                      jnp.maximum(p4[8 * i + 2 * j + 4], p4[8 * i + 2 * j + 5]))
          for i in range(2) for j in range(2)]
    p1 = jnp.maximum(jnp.maximum(p2[0], p2[1]), jnp.maximum(p2[2], p2[3]))
    pooled_ref[0] = jnp.concatenate([p1] + p2 + p4, axis=0)


def _head_kernel(flat_ref, wc_ref, bc_ref, logits_ref):
    flat = flat_ref[...]
    mu = jnp.mean(flat, axis=-1, keepdims=True)
    var = jnp.mean(jnp.square(flat - mu), axis=-1, keepdims=True)
    fn = ((flat - mu) * lax.rsqrt(var + LN_EPS)).astype(jnp.bfloat16)
    logits_ref[...] = (jnp.dot(fn, wc_ref[...],
                               preferred_element_type=jnp.float32)
                       + bc_ref[...])


def _fold_block(dw_w, dw_b, ln_g, ln_b, w1, b1, w2, b2, scale):
    w1f = ln_g.reshape(-1, 1) * w1
    b1f = b1 + ln_b @ w1
    w2f = w2 * scale
    b2f = b2 * scale
    return (dw_w, dw_b, w1f.astype(jnp.bfloat16), b1f,
            w2f.astype(jnp.bfloat16), b2f)


def kernel(x, dw_w_0, dw_b_0, ln_g_0, ln_b_0, w1_0, b1_0, w2_0, b2_0, scale_0,
           dw_w_1, dw_b_1, ln_g_1, ln_b_1, w1_1, b1_1, w2_1, b2_1, scale_1,
           dw_w_2, dw_b_2, ln_g_2, ln_b_2, w1_2, b1_2, w2_2, b2_2, scale_2,
           cls_ln_g, cls_ln_b, cls_w, cls_b):
    B, C, H, W = x.shape
    F = NPOOL * C
    nc = cls_w.shape[1]
    ncp = ((nc + 127) // 128) * 128

    xh = jnp.transpose(x, (0, 2, 3, 1)).astype(jnp.float32)

    flat_w = []
    flat_w += _fold_block(dw_w_0, dw_b_0, ln_g_0, ln_b_0, w1_0, b1_0, w2_0, b2_0, scale_0)
    flat_w += _fold_block(dw_w_1, dw_b_1, ln_g_1, ln_b_1, w1_1, b1_1, w2_1, b2_1, scale_1)
    flat_w += _fold_block(dw_w_2, dw_b_2, ln_g_2, ln_b_2, w1_2, b1_2, w2_2, b2_2, scale_2)

    hp = H + 2 * PAD
    wp = ((W + 2 * PAD + 7) // 8) * 8

    def whole(a):
        return pl.BlockSpec(a.shape, lambda b, n=a.ndim: (0,) * n)

    pooled = pl.pallas_call(
        functools.partial(_blocks_spp_kernel, num_blocks=3),
        out_shape=jax.ShapeDtypeStruct((B, NPOOL, C), jnp.float32),
        grid=(B,),
        in_specs=[pl.BlockSpec((1, H, W, C), lambda b: (b, 0, 0, 0))]
                 + [whole(a) for a in flat_w],
        out_specs=pl.BlockSpec((1, NPOOL, C), lambda b: (b, 0, 0)),
        scratch_shapes=[pltpu.VMEM((hp, wp, C), jnp.float32)],
        compiler_params=pltpu.CompilerParams(
            dimension_semantics=("parallel",),
            vmem_limit_bytes=48 * 1024 * 1024),
    )(xh, *flat_w)

    w_pm = cls_w.reshape(C, NPOOL, nc).transpose(1, 0, 2).reshape(F, nc)
    g_pm = cls_ln_g.reshape(C, NPOOL).T.reshape(F, 1)
    b_pm = cls_ln_b.reshape(C, NPOOL).T.reshape(1, F)
    wc = jnp.pad(g_pm * w_pm, ((0, 0), (0, ncp - nc))).astype(jnp.bfloat16)
    bc = jnp.pad(cls_b + b_pm @ w_pm, ((0, 0), (0, ncp - nc)))

    nh = ncp // 2
    logits = pl.pallas_call(
        _head_kernel,
        out_shape=jax.ShapeDtypeStruct((B, ncp), jnp.float32),
        grid=(2,),
        in_specs=[pl.BlockSpec((B, F), lambda g: (0, 0)),
                  pl.BlockSpec((F, nh), lambda g: (0, g)),
                  pl.BlockSpec((1, nh), lambda g: (0, g))],
        out_specs=pl.BlockSpec((B, nh), lambda g: (0, g)),
        compiler_params=pltpu.CompilerParams(
            dimension_semantics=("parallel",),
            vmem_limit_bytes=48 * 1024 * 1024),
    )(pooled.reshape(B, F), wc, bc)

    return logits[:, :nc], jnp.transpose(pooled, (0, 2, 1))


# speedup vs baseline: 1.0356x; 1.0126x over previous
import functools
import math

import jax
import jax.numpy as jnp
from jax import lax
from jax.experimental import pallas as pl
from jax.experimental.pallas import tpu as pltpu

KS = 7
PAD = 3
NPOOL = 21
LN_EPS = 1e-6
_GC = math.sqrt(2.0 / math.pi)


def _gelu(z):
    return 0.5 * z * (1.0 + jnp.tanh(_GC * (z + 0.044715 * z * z * z)))


def _one_block(y, band, dw_b, w1b, b1, w2b, b2):
    H, W, C = y.shape
    Wp = ((W + 2 * PAD + 7) // 8) * 8
    M = H * Wp
    R = (H + 2 * PAD) * Wp

    z_rows = jnp.zeros((PAD, Wp, C), jnp.float32)
    z_tail = jnp.zeros((PAD + 1, Wp, C), jnp.float32)
    z_l = jnp.zeros((H, PAD, C), jnp.float32)
    z_r = jnp.zeros((H, Wp - W - PAD, C), jnp.float32)
    mid = jnp.concatenate([z_l, y, z_r], axis=1)
    xp = jnp.concatenate([z_rows, mid, z_tail], axis=0)
    flat = xp.reshape((H + 2 * PAD + 1) * Wp, C)

    lhs = jnp.concatenate(
        [flat[dx:dx + R].astype(jnp.bfloat16) for dx in range(KS)], axis=1)

    acc = jnp.dot(lhs[:M], band[0], preferred_element_type=jnp.float32)
    for dy in range(1, KS):
        acc = acc + jnp.dot(lhs[dy * Wp:dy * Wp + M], band[dy],
                            preferred_element_type=jnp.float32)
    conv = acc.reshape(H, Wp, C)[:, :W, :] + dw_b

    mu = jnp.mean(conv, axis=-1, keepdims=True)
    var = jnp.mean(jnp.square(conv - mu), axis=-1, keepdims=True)
    xn = (conv - mu) * lax.rsqrt(var + LN_EPS)

    th = 8 if H % 8 == 0 else H
    outs = []
    for r0 in range(0, H, th):
        hb = xn[r0:r0 + th].reshape(th * W, C).astype(jnp.bfloat16)
        zz = jnp.dot(hb, w1b, preferred_element_type=jnp.float32) + b1
        zz = _gelu(zz).astype(jnp.bfloat16)
        o = jnp.dot(zz, w2b, preferred_element_type=jnp.float32) + b2
        outs.append(y[r0:r0 + th] + o.reshape(th, W, C))
    return outs[0] if len(outs) == 1 else jnp.concatenate(outs, axis=0)


def _blocks_spp_kernel(x_ref, *refs, num_blocks):
    blk = [refs[6 * b:6 * (b + 1)] for b in range(num_blocks)]
    pooled_ref = refs[6 * num_blocks]

    y = x_ref[0]
    for b in range(num_blocks):
        band, dw_b, w1b, b1, w2b, b2 = (r[...] for r in blk[b])
        y = _one_block(y, band, dw_b, w1b, b1, w2b, b2)

    H, W, C = y.shape
    bh, bw = H // 4, W // 4
    p4 = []
    for j in range(4):
        colmax = jnp.max(y[:, j * bw:(j + 1) * bw, :], axis=1)
        for i in range(4):
            p4.append(jnp.max(colmax[i * bh:(i + 1) * bh], axis=0,
                              keepdims=True))
    p4 = [p4[j * 4 + i] for i in range(4) for j in range(4)]
    p2 = [jnp.maximum(jnp.maximum(p4[8 * i + 2 * j], p4[8 * i + 2 * j + 1]),
                      jnp.maximum(p4[8 * i + 2 * j + 4], p4[8 * i + 2 * j + 5]))
          for i in range(2) for j in range(2)]
    p1 = jnp.maximum(jnp.maximum(p2[0], p2[1]), jnp.maximum(p2[2], p2[3]))
    pooled_ref[0] = jnp.concatenate([p1] + p2 + p4, axis=0)


def _head_kernel(flat_ref, wc_ref, bc_ref, logits_ref):
    flat = flat_ref[...]
    mu = jnp.mean(flat, axis=-1, keepdims=True)
    var = jnp.mean(jnp.square(flat - mu), axis=-1, keepdims=True)
    fn = ((flat - mu) * lax.rsqrt(var + LN_EPS)).astype(jnp.bfloat16)
    logits_ref[...] = (jnp.dot(fn, wc_ref[...],
                               preferred_element_type=jnp.float32)
                       + bc_ref[...])


def _fold_block(dw_w, dw_b, ln_g, ln_b, w1, b1, w2, b2, scale):
    C = dw_w.shape[-1]
    eye = jnp.eye(C, dtype=jnp.float32)
    band = jnp.stack([
        jnp.concatenate([eye * dw_w[dy, dx][None, :] for dx in range(KS)],
                        axis=0)
        for dy in range(KS)])
    w1f = ln_g.reshape(-1, 1) * w1
    b1f = b1 + ln_b @ w1
    w2f = w2 * scale
    b2f = b2 * scale
    return (band.astype(jnp.bfloat16), dw_b, w1f.astype(jnp.bfloat16), b1f,
            w2f.astype(jnp.bfloat16), b2f)


def kernel(x, dw_w_0, dw_b_0, ln_g_0, ln_b_0, w1_0, b1_0, w2_0, b2_0, scale_0,
           dw_w_1, dw_b_1, ln_g_1, ln_b_1, w1_1, b1_1, w2_1, b2_1, scale_1,
           dw_w_2, dw_b_2, ln_g_2, ln_b_2, w1_2, b1_2, w2_2, b2_2, scale_2,
           cls_ln_g, cls_ln_b, cls_w, cls_b):
    B, C, H, W = x.shape
    F = NPOOL * C
    nc = cls_w.shape[1]
    ncp = ((nc + 127) // 128) * 128

    xh = jnp.transpose(x, (0, 2, 3, 1)).astype(jnp.float32)

    flat_w = []
    flat_w += _fold_block(dw_w_0, dw_b_0, ln_g_0, ln_b_0, w1_0, b1_0, w2_0, b2_0, scale_0)
    flat_w += _fold_block(dw_w_1, dw_b_1, ln_g_1, ln_b_1, w1_1, b1_1, w2_1, b2_1, scale_1)
    flat_w += _fold_block(dw_w_2, dw_b_2, ln_g_2, ln_b_2, w1_2, b1_2, w2_2, b2_2, scale_2)

    def whole(a):
        return pl.BlockSpec(a.shape, lambda b, n=a.ndim: (0,) * n)

    pooled = pl.pallas_call(
        functools.partial(_blocks_spp_kernel, num_blocks=3),
        out_shape=jax.ShapeDtypeStruct((B, NPOOL, C), jnp.float32),
        grid=(B,),
        in_specs=[pl.BlockSpec((1, H, W, C), lambda b: (b, 0, 0, 0))]
                 + [whole(a) for a in flat_w],
        out_specs=pl.BlockSpec((1, NPOOL, C), lambda b: (b, 0, 0)),
        compiler_params=pltpu.CompilerParams(
            dimension_semantics=("parallel",),
            vmem_limit_bytes=48 * 1024 * 1024),
    )(xh, *flat_w)

    w_pm = cls_w.reshape(C, NPOOL, nc).transpose(1, 0, 2).reshape(F, nc)
    g_pm = cls_ln_g.reshape(C, NPOOL).T.reshape(F, 1)
    b_pm = cls_ln_b.reshape(C, NPOOL).T.reshape(1, F)
    wc = jnp.pad(g_pm * w_pm, ((0, 0), (0, ncp - nc))).astype(jnp.bfloat16)
    bc = jnp.pad(cls_b + b_pm @ w_pm, ((0, 0), (0, ncp - nc)))

    nh = ncp // 2
    logits = pl.pallas_call(
        _head_kernel,
        out_shape=jax.ShapeDtypeStruct((B, ncp), jnp.float32),
        grid=(2,),
        in_specs=[pl.BlockSpec((B, F), lambda g: (0, 0)),
                  pl.BlockSpec((F, nh), lambda g: (0, g)),
                  pl.BlockSpec((1, nh), lambda g: (0, g))],
        out_specs=pl.BlockSpec((B, nh), lambda g: (0, g)),
        compiler_params=pltpu.CompilerParams(
            dimension_semantics=("parallel",),
            vmem_limit_bytes=48 * 1024 * 1024),
    )(pooled.reshape(B, F), wc, bc)

    return logits[:, :nc], jnp.transpose(pooled, (0, 2, 1))


# speedup vs baseline: 1.1540x; 1.1143x over previous
import functools
import math

import jax
import jax.numpy as jnp
from jax import lax
from jax.experimental import pallas as pl
from jax.experimental.pallas import tpu as pltpu

KS = 7
PAD = 3
NPOOL = 21
LN_EPS = 1e-6
_GC = math.sqrt(2.0 / math.pi)


def _gelu(z):
    return 0.5 * z * (1.0 + jnp.tanh(_GC * (z + 0.044715 * z * z * z)))


def _one_block(y, band, dw_b, w1b, b1, w2b, b2):
    H, W, C = y.shape
    Wp = ((W + 2 * PAD + 7) // 8) * 8
    M = H * Wp
    R = (H + 2 * PAD) * Wp

    z_rows = jnp.zeros((PAD, Wp, C), jnp.float32)
    z_tail = jnp.zeros((PAD + 1, Wp, C), jnp.float32)
    z_l = jnp.zeros((H, PAD, C), jnp.float32)
    z_r = jnp.zeros((H, Wp - W - PAD, C), jnp.float32)
    mid = jnp.concatenate([z_l, y, z_r], axis=1)
    xp = jnp.concatenate([z_rows, mid, z_tail], axis=0)
    flat = xp.reshape((H + 2 * PAD + 1) * Wp, C)

    lhs = jnp.concatenate(
        [flat[dx:dx + R].astype(jnp.bfloat16) for dx in range(KS)], axis=1)

    acc = jnp.dot(lhs[:M], band[0], preferred_element_type=jnp.float32)
    for dy in range(1, KS):
        acc = acc + jnp.dot(lhs[dy * Wp:dy * Wp + M], band[dy],
                            preferred_element_type=jnp.float32)
    conv = acc.reshape(H, Wp, C)[:, :W, :] + dw_b

    mu = jnp.mean(conv, axis=-1, keepdims=True)
    var = jnp.mean(jnp.square(conv - mu), axis=-1, keepdims=True)
    xn = (conv - mu) * lax.rsqrt(var + LN_EPS)

    th = 8 if H % 8 == 0 else H
    outs = []
    for r0 in range(0, H, th):
        hb = xn[r0:r0 + th].reshape(th * W, C).astype(jnp.bfloat16)
        zz = jnp.dot(hb, w1b, preferred_element_type=jnp.float32) + b1
        zz = _gelu(zz).astype(jnp.bfloat16)
        o = jnp.dot(zz, w2b, preferred_element_type=jnp.float32) + b2
        outs.append(y[r0:r0 + th] + o.reshape(th, W, C))
    return outs[0] if len(outs) == 1 else jnp.concatenate(outs, axis=0)


def _spp(y):
    H, W, C = y.shape
    bh, bw = H // 4, W // 4
    p4 = []
    for j in range(4):
        colmax = jnp.max(y[:, j * bw:(j + 1) * bw, :], axis=1)
        for i in range(4):
            p4.append(jnp.max(colmax[i * bh:(i + 1) * bh], axis=0,
                              keepdims=True))
    p4 = [p4[j * 4 + i] for i in range(4) for j in range(4)]
    p2 = [jnp.maximum(jnp.maximum(p4[8 * i + 2 * j], p4[8 * i + 2 * j + 1]),
                      jnp.maximum(p4[8 * i + 2 * j + 4], p4[8 * i + 2 * j + 5]))
          for i in range(2) for j in range(2)]
    p1 = jnp.maximum(jnp.maximum(p2[0], p2[1]), jnp.maximum(p2[2], p2[3]))
    return jnp.concatenate([p1] + p2 + p4, axis=0)


def _blocks_spp_kernel(x_ref, *refs, num_blocks, imgs):
    blk = [refs[6 * b:6 * (b + 1)] for b in range(num_blocks)]
    pooled_ref = refs[6 * num_blocks]

    ys = [x_ref[i] for i in range(imgs)]
    for b in range(num_blocks):
        band, dw_b, w1b, b1, w2b, b2 = (r[...] for r in blk[b])
        ys = [_one_block(y, band, dw_b, w1b, b1, w2b, b2) for y in ys]
    for i in range(imgs):
        pooled_ref[i] = _spp(ys[i])


def _head_kernel(flat_ref, wc_ref, bc_ref, logits_ref):
    flat = flat_ref[...]
    mu = jnp.mean(flat, axis=-1, keepdims=True)
    var = jnp.mean(jnp.square(flat - mu), axis=-1, keepdims=True)
    fn = ((flat - mu) * lax.rsqrt(var + LN_EPS)).astype(jnp.bfloat16)
    logits_ref[...] = (jnp.dot(fn, wc_ref[...],
                               preferred_element_type=jnp.float32)
                       + bc_ref[...])


def _fold_block(dw_w, dw_b, ln_g, ln_b, w1, b1, w2, b2, scale):
    C = dw_w.shape[-1]
    eye = jnp.eye(C, dtype=jnp.float32)
    band = jnp.stack([
        jnp.concatenate([eye * dw_w[dy, dx][None, :] for dx in range(KS)],
                        axis=0)
        for dy in range(KS)])
    w1f = ln_g.reshape(-1, 1) * w1
    b1f = b1 + ln_b @ w1
    w2f = w2 * scale
    b2f = b2 * scale
    return (band.astype(jnp.bfloat16), dw_b, w1f.astype(jnp.bfloat16), b1f,
            w2f.astype(jnp.bfloat16), b2f)


def kernel(x, dw_w_0, dw_b_0, ln_g_0, ln_b_0, w1_0, b1_0, w2_0, b2_0, scale_0,
           dw_w_1, dw_b_1, ln_g_1, ln_b_1, w1_1, b1_1, w2_1, b2_1, scale_1,
           dw_w_2, dw_b_2, ln_g_2, ln_b_2, w1_2, b1_2, w2_2, b2_2, scale_2,
           cls_ln_g, cls_ln_b, cls_w, cls_b):
    B, C, H, W = x.shape
    F = NPOOL * C
    nc = cls_w.shape[1]
    ncp = ((nc + 127) // 128) * 128

    xh = jnp.transpose(x, (0, 2, 3, 1)).astype(jnp.float32)

    flat_w = []
    flat_w += _fold_block(dw_w_0, dw_b_0, ln_g_0, ln_b_0, w1_0, b1_0, w2_0, b2_0, scale_0)
    flat_w += _fold_block(dw_w_1, dw_b_1, ln_g_1, ln_b_1, w1_1, b1_1, w2_1, b2_1, scale_1)
    flat_w += _fold_block(dw_w_2, dw_b_2, ln_g_2, ln_b_2, w1_2, b1_2, w2_2, b2_2, scale_2)

    def whole(a):
        return pl.BlockSpec(a.shape, lambda b, n=a.ndim: (0,) * n)

    gi = 2 if B % 2 == 0 else 1
    pooled = pl.pallas_call(
        functools.partial(_blocks_spp_kernel, num_blocks=3, imgs=gi),
        out_shape=jax.ShapeDtypeStruct((B, NPOOL, C), jnp.float32),
        grid=(B // gi,),
        in_specs=[pl.BlockSpec((gi, H, W, C), lambda b: (b, 0, 0, 0))]
                 + [whole(a) for a in flat_w],
        out_specs=pl.BlockSpec((gi, NPOOL, C), lambda b: (b, 0, 0)),
        compiler_params=pltpu.CompilerParams(
            dimension_semantics=("parallel",),
            vmem_limit_bytes=56 * 1024 * 1024),
    )(xh, *flat_w)

    w_pm = cls_w.reshape(C, NPOOL, nc).transpose(1, 0, 2).reshape(F, nc)
    g_pm = cls_ln_g.reshape(C, NPOOL).T.reshape(F, 1)
    b_pm = cls_ln_b.reshape(C, NPOOL).T.reshape(1, F)
    wc = jnp.pad(g_pm * w_pm, ((0, 0), (0, ncp - nc))).astype(jnp.bfloat16)
    bc = jnp.pad(cls_b + b_pm @ w_pm, ((0, 0), (0, ncp - nc)))

    nh = ncp // 2
    logits = pl.pallas_call(
        _head_kernel,
        out_shape=jax.ShapeDtypeStruct((B, ncp), jnp.float32),
        grid=(2,),
        in_specs=[pl.BlockSpec((B, F), lambda g: (0, 0)),
                  pl.BlockSpec((F, nh), lambda g: (0, g)),
                  pl.BlockSpec((1, nh), lambda g: (0, g))],
        out_specs=pl.BlockSpec((B, nh), lambda g: (0, g)),
        compiler_params=pltpu.CompilerParams(
            dimension_semantics=("parallel",),
            vmem_limit_bytes=48 * 1024 * 1024),
    )(pooled.reshape(B, F), wc, bc)

    return logits[:, :nc], jnp.transpose(pooled, (0, 2, 1))


# speedup vs baseline: 1.2134x; 1.0515x over previous
import functools
import math

import jax
import jax.numpy as jnp
from jax import lax
from jax.experimental import pallas as pl
from jax.experimental.pallas import tpu as pltpu

KS = 7
PAD = 3
NPOOL = 21
LN_EPS = 1e-6
_GC = math.sqrt(2.0 / math.pi)


def _gelu(z):
    return 0.5 * z * (1.0 + jnp.tanh(_GC * (z + 0.044715 * z * z * z)))


def _one_block(y, band, dw_b, w1b, b1, w2b, b2):
    H, W, C = y.shape
    Wp = ((W + 2 * PAD + 7) // 8) * 8
    M = H * Wp
    R = (H + 2 * PAD) * Wp

    z_rows = jnp.zeros((PAD, Wp, C), jnp.float32)
    z_tail = jnp.zeros((PAD + 1, Wp, C), jnp.float32)
    z_l = jnp.zeros((H, PAD, C), jnp.float32)
    z_r = jnp.zeros((H, Wp - W - PAD, C), jnp.float32)
    mid = jnp.concatenate([z_l, y, z_r], axis=1)
    xp = jnp.concatenate([z_rows, mid, z_tail], axis=0)
    flat = xp.reshape((H + 2 * PAD + 1) * Wp, C)

    lhs = jnp.concatenate(
        [flat[dx:dx + R].astype(jnp.bfloat16) for dx in range(KS)], axis=1)

    acc = jnp.dot(lhs[:M], band[0], preferred_element_type=jnp.float32)
    for dy in range(1, KS):
        acc = acc + jnp.dot(lhs[dy * Wp:dy * Wp + M], band[dy],
                            preferred_element_type=jnp.float32)
    conv = acc.reshape(H, Wp, C)[:, :W, :] + dw_b

    mu = jnp.mean(conv, axis=-1, keepdims=True)
    var = jnp.mean(jnp.square(conv - mu), axis=-1, keepdims=True)
    xn = (conv - mu) * lax.rsqrt(var + LN_EPS)

    th = 8 if H % 8 == 0 else H
    outs = []
    for r0 in range(0, H, th):
        hb = xn[r0:r0 + th].reshape(th * W, C).astype(jnp.bfloat16)
        zz = jnp.dot(hb, w1b, preferred_element_type=jnp.float32) + b1
        zz = _gelu(zz).astype(jnp.bfloat16)
        o = jnp.dot(zz, w2b, preferred_element_type=jnp.float32) + b2
        outs.append(y[r0:r0 + th] + o.reshape(th, W, C))
    return outs[0] if len(outs) == 1 else jnp.concatenate(outs, axis=0)


def _spp(y):
    H, W, C = y.shape
    bh, bw = H // 4, W // 4
    p4 = []
    for j in range(4):
        colmax = jnp.max(y[:, j * bw:(j + 1) * bw, :], axis=1)
        for i in range(4):
            p4.append(jnp.max(colmax[i * bh:(i + 1) * bh], axis=0,
                              keepdims=True))
    p4 = [p4[j * 4 + i] for i in range(4) for j in range(4)]
    p2 = [jnp.maximum(jnp.maximum(p4[8 * i + 2 * j], p4[8 * i + 2 * j + 1]),
                      jnp.maximum(p4[8 * i + 2 * j + 4], p4[8 * i + 2 * j + 5]))
          for i in range(2) for j in range(2)]
    p1 = jnp.maximum(jnp.maximum(p2[0], p2[1]), jnp.maximum(p2[2], p2[3]))
    return jnp.concatenate([p1] + p2 + p4, axis=0)


def _blocks_spp_kernel(x_ref, *refs, num_blocks, imgs):
    blk = [refs[6 * b:6 * (b + 1)] for b in range(num_blocks)]
    pooled_ref = refs[6 * num_blocks]

    ys = [x_ref[i] for i in range(imgs)]
    for b in range(num_blocks):
        band, dw_b, w1b, b1, w2b, b2 = (r[...] for r in blk[b])
        ys = [_one_block(y, band, dw_b, w1b, b1, w2b, b2) for y in ys]
    for i in range(imgs):
        pooled_ref[i] = _spp(ys[i])


def _head_kernel(flat_ref, wc_ref, bc_ref, logits_ref):
    flat = flat_ref[...]
    mu = jnp.mean(flat, axis=-1, keepdims=True)
    var = jnp.mean(jnp.square(flat - mu), axis=-1, keepdims=True)
    fn = ((flat - mu) * lax.rsqrt(var + LN_EPS)).astype(jnp.bfloat16)
    logits_ref[...] = (jnp.dot(fn, wc_ref[...],
                               preferred_element_type=jnp.float32)
                       + bc_ref[...])


def _fold_block(dw_w, dw_b, ln_g, ln_b, w1, b1, w2, b2, scale):
    C = dw_w.shape[-1]
    eye = jnp.eye(C, dtype=jnp.float32)
    band = jnp.stack([
        jnp.concatenate([eye * dw_w[dy, dx][None, :] for dx in range(KS)],
                        axis=0)
        for dy in range(KS)])
    w1f = ln_g.reshape(-1, 1) * w1
    b1f = b1 + ln_b @ w1
    w2f = w2 * scale
    b2f = b2 * scale
    return (band.astype(jnp.bfloat16), dw_b, w1f.astype(jnp.bfloat16), b1f,
            w2f.astype(jnp.bfloat16), b2f)


def kernel(x, dw_w_0, dw_b_0, ln_g_0, ln_b_0, w1_0, b1_0, w2_0, b2_0, scale_0,
           dw_w_1, dw_b_1, ln_g_1, ln_b_1, w1_1, b1_1, w2_1, b2_1, scale_1,
           dw_w_2, dw_b_2, ln_g_2, ln_b_2, w1_2, b1_2, w2_2, b2_2, scale_2,
           cls_ln_g, cls_ln_b, cls_w, cls_b):
    B, C, H, W = x.shape
    F = NPOOL * C
    nc = cls_w.shape[1]
    ncp = ((nc + 127) // 128) * 128

    xh = jnp.transpose(x, (0, 2, 3, 1)).astype(jnp.float32)

    flat_w = []
    flat_w += _fold_block(dw_w_0, dw_b_0, ln_g_0, ln_b_0, w1_0, b1_0, w2_0, b2_0, scale_0)
    flat_w += _fold_block(dw_w_1, dw_b_1, ln_g_1, ln_b_1, w1_1, b1_1, w2_1, b2_1, scale_1)
    flat_w += _fold_block(dw_w_2, dw_b_2, ln_g_2, ln_b_2, w1_2, b1_2, w2_2, b2_2, scale_2)

    def whole(a):
        return pl.BlockSpec(a.shape, lambda b, n=a.ndim: (0,) * n)

    gi = 4 if B % 4 == 0 else (2 if B % 2 == 0 else 1)
    pooled = pl.pallas_call(
        functools.partial(_blocks_spp_kernel, num_blocks=3, imgs=gi),
        out_shape=jax.ShapeDtypeStruct((B, NPOOL, C), jnp.float32),
        grid=(B // gi,),
        in_specs=[pl.BlockSpec((gi, H, W, C), lambda b: (b, 0, 0, 0))]
                 + [whole(a) for a in flat_w],
        out_specs=pl.BlockSpec((gi, NPOOL, C), lambda b: (b, 0, 0)),
        compiler_params=pltpu.CompilerParams(
            dimension_semantics=("parallel",),
            vmem_limit_bytes=56 * 1024 * 1024),
    )(xh, *flat_w)

    w_pm = cls_w.reshape(C, NPOOL, nc).transpose(1, 0, 2).reshape(F, nc)
    g_pm = cls_ln_g.reshape(C, NPOOL).T.reshape(F, 1)
    b_pm = cls_ln_b.reshape(C, NPOOL).T.reshape(1, F)
    wc = jnp.pad(g_pm * w_pm, ((0, 0), (0, ncp - nc))).astype(jnp.bfloat16)
    bc = jnp.pad(cls_b + b_pm @ w_pm, ((0, 0), (0, ncp - nc)))

    nh = ncp // 2
    logits = pl.pallas_call(
        _head_kernel,
        out_shape=jax.ShapeDtypeStruct((B, ncp), jnp.float32),
        grid=(2,),
        in_specs=[pl.BlockSpec((B, F), lambda g: (0, 0)),
                  pl.BlockSpec((F, nh), lambda g: (0, g)),
                  pl.BlockSpec((1, nh), lambda g: (0, g))],
        out_specs=pl.BlockSpec((B, nh), lambda g: (0, g)),
        compiler_params=pltpu.CompilerParams(
            dimension_semantics=("parallel",),
            vmem_limit_bytes=48 * 1024 * 1024),
    )(pooled.reshape(B, F), wc, bc)

    return logits[:, :nc], jnp.transpose(pooled, (0, 2, 1))


# speedup vs baseline: 1.2793x; 1.0543x over previous
import functools
import math

import jax
import jax.numpy as jnp
from jax import lax
from jax.experimental import pallas as pl
from jax.experimental.pallas import tpu as pltpu

KS = 7
PAD = 3
NPOOL = 21
LN_EPS = 1e-6
_GC = math.sqrt(2.0 / math.pi)


def _gelu(z):
    return 0.5 * z * (1.0 + jnp.tanh(_GC * (z + 0.044715 * z * z * z)))


def _one_block(y, band, dw_w, dw_b, w1b, b1, w2b, b2, on_mxu):
    H, W, C = y.shape
    Wp = ((W + 2 * PAD + 7) // 8) * 8
    M = H * Wp
    R = (H + 2 * PAD) * Wp

    z_rows = jnp.zeros((PAD, Wp, C), jnp.float32)
    z_tail = jnp.zeros((PAD + 1, Wp, C), jnp.float32)
    z_l = jnp.zeros((H, PAD, C), jnp.float32)
    z_r = jnp.zeros((H, Wp - W - PAD, C), jnp.float32)
    mid = jnp.concatenate([z_l, y, z_r], axis=1)
    xp = jnp.concatenate([z_rows, mid, z_tail], axis=0)

    if on_mxu:
        flat = xp.reshape((H + 2 * PAD + 1) * Wp, C)
        lhs = jnp.concatenate(
            [flat[dx:dx + R].astype(jnp.bfloat16) for dx in range(KS)],
            axis=1)
        acc = jnp.dot(lhs[:M], band[0], preferred_element_type=jnp.float32)
        for dy in range(1, KS):
            acc = acc + jnp.dot(lhs[dy * Wp:dy * Wp + M], band[dy],
                                preferred_element_type=jnp.float32)
        conv = acc.reshape(H, Wp, C)[:, :W, :] + dw_b
    else:
        acc = jnp.zeros((H, W, C), jnp.float32)
        for dx in range(KS):
            win = xp[:H + 2 * PAD, dx:dx + W, :]
            for dy in range(KS):
                acc = acc + win[dy:dy + H] * dw_w[dy, dx]
        conv = acc + dw_b

    mu = jnp.mean(conv, axis=-1, keepdims=True)
    var = jnp.mean(jnp.square(conv - mu), axis=-1, keepdims=True)
    xn = (conv - mu) * lax.rsqrt(var + LN_EPS)

    th = 8 if H % 8 == 0 else H
    outs = []
    for r0 in range(0, H, th):
        hb = xn[r0:r0 + th].reshape(th * W, C).astype(jnp.bfloat16)
        zz = jnp.dot(hb, w1b, preferred_element_type=jnp.float32) + b1
        zz = _gelu(zz).astype(jnp.bfloat16)
        o = jnp.dot(zz, w2b, preferred_element_type=jnp.float32) + b2
        outs.append(y[r0:r0 + th] + o.reshape(th, W, C))
    return outs[0] if len(outs) == 1 else jnp.concatenate(outs, axis=0)


def _spp(y):
    H, W, C = y.shape
    bh, bw = H // 4, W // 4
    p4 = []
    for j in range(4):
        colmax = jnp.max(y[:, j * bw:(j + 1) * bw, :], axis=1)
        for i in range(4):
            p4.append(jnp.max(colmax[i * bh:(i + 1) * bh], axis=0,
                              keepdims=True))
    p4 = [p4[j * 4 + i] for i in range(4) for j in range(4)]
    p2 = [jnp.maximum(jnp.maximum(p4[8 * i + 2 * j], p4[8 * i + 2 * j + 1]),
                      jnp.maximum(p4[8 * i + 2 * j + 4], p4[8 * i + 2 * j + 5]))
          for i in range(2) for j in range(2)]
    p1 = jnp.maximum(jnp.maximum(p2[0], p2[1]), jnp.maximum(p2[2], p2[3]))
    return jnp.concatenate([p1] + p2 + p4, axis=0)


def _blocks_spp_kernel(x_ref, *refs, num_blocks, imgs, n_mxu):
    blk = [refs[7 * b:7 * (b + 1)] for b in range(num_blocks)]
    pooled_ref = refs[7 * num_blocks]

    ys = [x_ref[i] for i in range(imgs)]
    for b in range(num_blocks):
        band, dw_w, dw_b, w1b, b1, w2b, b2 = (r[...] for r in blk[b])
        ys = [_one_block(y, band, dw_w, dw_b, w1b, b1, w2b, b2, i < n_mxu)
              for i, y in enumerate(ys)]
    for i in range(imgs):
        pooled_ref[i] = _spp(ys[i])


def _head_kernel(flat_ref, wc_ref, bc_ref, logits_ref):
    flat = flat_ref[...]
    mu = jnp.mean(flat, axis=-1, keepdims=True)
    var = jnp.mean(jnp.square(flat - mu), axis=-1, keepdims=True)
    fn = ((flat - mu) * lax.rsqrt(var + LN_EPS)).astype(jnp.bfloat16)
    logits_ref[...] = (jnp.dot(fn, wc_ref[...],
                               preferred_element_type=jnp.float32)
                       + bc_ref[...])


def _fold_block(dw_w, dw_b, ln_g, ln_b, w1, b1, w2, b2, scale):
    C = dw_w.shape[-1]
    eye = jnp.eye(C, dtype=jnp.float32)
    band = jnp.stack([
        jnp.concatenate([eye * dw_w[dy, dx][None, :] for dx in range(KS)],
                        axis=0)
        for dy in range(KS)])
    w1f = ln_g.reshape(-1, 1) * w1
    b1f = b1 + ln_b @ w1
    w2f = w2 * scale
    b2f = b2 * scale
    return (band.astype(jnp.bfloat16), dw_w, dw_b,
            w1f.astype(jnp.bfloat16), b1f, w2f.astype(jnp.bfloat16), b2f)


def kernel(x, dw_w_0, dw_b_0, ln_g_0, ln_b_0, w1_0, b1_0, w2_0, b2_0, scale_0,
           dw_w_1, dw_b_1, ln_g_1, ln_b_1, w1_1, b1_1, w2_1, b2_1, scale_1,
           dw_w_2, dw_b_2, ln_g_2, ln_b_2, w1_2, b1_2, w2_2, b2_2, scale_2,
           cls_ln_g, cls_ln_b, cls_w, cls_b):
    B, C, H, W = x.shape
    F = NPOOL * C
    nc = cls_w.shape[1]
    ncp = ((nc + 127) // 128) * 128

    xh = jnp.transpose(x, (0, 2, 3, 1)).astype(jnp.float32)

    flat_w = []
    flat_w += _fold_block(dw_w_0, dw_b_0, ln_g_0, ln_b_0, w1_0, b1_0, w2_0, b2_0, scale_0)
    flat_w += _fold_block(dw_w_1, dw_b_1, ln_g_1, ln_b_1, w1_1, b1_1, w2_1, b2_1, scale_1)
    flat_w += _fold_block(dw_w_2, dw_b_2, ln_g_2, ln_b_2, w1_2, b1_2, w2_2, b2_2, scale_2)

    def whole(a):
        return pl.BlockSpec(a.shape, lambda b, n=a.ndim: (0,) * n)

    gi = 4 if B % 4 == 0 else (2 if B % 2 == 0 else 1)
    nm = 3 if gi == 4 else gi
    pooled = pl.pallas_call(
        functools.partial(_blocks_spp_kernel, num_blocks=3, imgs=gi, n_mxu=nm),
        out_shape=jax.ShapeDtypeStruct((B, NPOOL, C), jnp.float32),
        grid=(B // gi,),
        in_specs=[pl.BlockSpec((gi, H, W, C), lambda b: (b, 0, 0, 0))]
                 + [whole(a) for a in flat_w],
        out_specs=pl.BlockSpec((gi, NPOOL, C), lambda b: (b, 0, 0)),
        compiler_params=pltpu.CompilerParams(
            dimension_semantics=("parallel",),
            vmem_limit_bytes=56 * 1024 * 1024),
    )(xh, *flat_w)

    w_pm = cls_w.reshape(C, NPOOL, nc).transpose(1, 0, 2).reshape(F, nc)
    g_pm = cls_ln_g.reshape(C, NPOOL).T.reshape(F, 1)
    b_pm = cls_ln_b.reshape(C, NPOOL).T.reshape(1, F)
    wc = jnp.pad(g_pm * w_pm, ((0, 0), (0, ncp - nc))).astype(jnp.bfloat16)
    bc = jnp.pad(cls_b + b_pm @ w_pm, ((0, 0), (0, ncp - nc)))

    nh = ncp // 2
    logits = pl.pallas_call(
        _head_kernel,
        out_shape=jax.ShapeDtypeStruct((B, ncp), jnp.float32),
        grid=(2,),
        in_specs=[pl.BlockSpec((B, F), lambda g: (0, 0)),
                  pl.BlockSpec((F, nh), lambda g: (0, g)),
                  pl.BlockSpec((1, nh), lambda g: (0, g))],
        out_specs=pl.BlockSpec((B, nh), lambda g: (0, g)),
        compiler_params=pltpu.CompilerParams(
            dimension_semantics=("parallel",),
            vmem_limit_bytes=48 * 1024 * 1024),
    )(pooled.reshape(B, F), wc, bc)

    return logits[:, :nc], jnp.transpose(pooled, (0, 2, 1))
